# baseline (speedup 1.0000x reference)
import functools

import jax
import jax.numpy as jnp
from jax import lax
from jax.experimental import pallas as pl
from jax.experimental.pallas import tpu as pltpu
from jax.experimental.pallas import tpu_sc as plsc

HIDDEN_DIM = 2048
NUM_CHOICES = 64
BLOCK_M = 1024
N_TOK = 16384

_info = plsc.get_sparse_core_info()
_NC, _NS = _info.num_cores, _info.num_subcores
_NW = _NC * _NS
_B_PER_W = N_TOK // _NW
_GATHER_CHUNK = 128


def _argmax_kernel(x_ref, w_ref, b_ref, o_ref):
    logits_t = jax.lax.dot_general(
        w_ref[...], x_ref[...],
        dimension_numbers=(((1,), (1,)), ((), ())),
        preferred_element_type=jnp.float32,
    )
    logits_t = logits_t + b_ref[...]
    col_max = jnp.max(logits_t, axis=0, keepdims=True)
    row = jax.lax.broadcasted_iota(jnp.int32, logits_t.shape, 0)
    cand = jnp.where(logits_t == col_max, row, NUM_CHOICES)
    idx = jnp.min(cand, axis=0, keepdims=True)
    o_ref[...] = jnp.broadcast_to(idx, (8, idx.shape[1]))


def _tc_argmax(x, W, b):
    n = x.shape[0]
    b2 = b.reshape(NUM_CHOICES, 1)
    return pl.pallas_call(
        _argmax_kernel,
        grid=(n // BLOCK_M,),
        in_specs=[
            pl.BlockSpec((BLOCK_M, HIDDEN_DIM), lambda i: (i, 0)),
            pl.BlockSpec((NUM_CHOICES, HIDDEN_DIM), lambda i: (0, 0)),
            pl.BlockSpec((NUM_CHOICES, 1), lambda i: (0, 0)),
        ],
        out_specs=pl.BlockSpec((8, BLOCK_M), lambda i: (0, i)),
        out_shape=jax.ShapeDtypeStruct((8, n), jnp.int32),
        compiler_params=pltpu.CompilerParams(
            dimension_semantics=("arbitrary",),
        ),
    )(x, W, b2)


@functools.partial(
    pl.kernel,
    mesh=plsc.VectorSubcoreMesh(core_axis_name="c", subcore_axis_name="s"),
    out_type=jax.ShapeDtypeStruct((NUM_CHOICES, N_TOK), jnp.float32),
    scratch_types=[
        pltpu.VMEM((_B_PER_W,), jnp.int32),
        pltpu.VMEM((NUM_CHOICES, _B_PER_W), jnp.float32),
    ],
)
def _sc_onehot(idx_hbm, out_hbm, idx_v, buf_v):
    wid = lax.axis_index("s") * _NC + lax.axis_index("c")
    base = wid * _B_PER_W
    pltpu.sync_copy(idx_hbm.at[pl.ds(base, _B_PER_W)], idx_v)

    @pl.loop(0, _B_PER_W // 16)
    def _chunk(j):
        v = idx_v[pl.ds(j * 16, 16)]
        for c in range(NUM_CHOICES):
            buf_v[c, pl.ds(j * 16, 16)] = jnp.where(
                v == c, jnp.float32(1.0), jnp.float32(0.0)
            )

    pltpu.sync_copy(buf_v, out_hbm.at[:, pl.ds(base, _B_PER_W)])


def kernel(x, W, b):
    idx8 = _tc_argmax(x, W, b)
    idx1d = idx8[0]
    return _sc_onehot(idx1d).T


# speedup vs baseline: 1.3855x; 1.3855x over previous
import jax
import jax.numpy as jnp
from jax.experimental import pallas as pl
from jax.experimental.pallas import tpu as pltpu

HIDDEN_DIM = 2048
NUM_CHOICES = 64
BLOCK_M = 1024


def _gate_kernel(x_ref, w_hbm, b_ref, o_ref, w_vmem, sem):
    @pl.when(pl.program_id(0) == 0)
    def _fetch_w():
        cp = pltpu.make_async_copy(w_hbm, w_vmem, sem)
        cp.start()
        cp.wait()

    logits_t = jax.lax.dot_general(
        w_vmem[...], x_ref[...],
        dimension_numbers=(((1,), (1,)), ((), ())),
        preferred_element_type=jnp.float32,
    )
    logits_t = logits_t + b_ref[...]
    col_max = jnp.max(logits_t, axis=0, keepdims=True)
    row = jax.lax.broadcasted_iota(jnp.int32, logits_t.shape, 0)
    cand = jnp.where(logits_t == col_max, row, NUM_CHOICES)
    idx = jnp.min(cand, axis=0, keepdims=True)
    o_ref[...] = (row == idx).astype(o_ref.dtype)


def kernel(x, W, b):
    n = x.shape[0]
    b2 = b.reshape(NUM_CHOICES, 1)
    out_t = pl.pallas_call(
        _gate_kernel,
        grid=(n // BLOCK_M,),
        in_specs=[
            pl.BlockSpec((BLOCK_M, HIDDEN_DIM), lambda i: (i, 0)),
            pl.BlockSpec(memory_space=pltpu.MemorySpace.HBM),
            pl.BlockSpec((NUM_CHOICES, 1), lambda i: (0, 0)),
        ],
        out_specs=pl.BlockSpec((NUM_CHOICES, BLOCK_M), lambda i: (0, i)),
        out_shape=jax.ShapeDtypeStruct((NUM_CHOICES, n), x.dtype),
        scratch_shapes=[
            pltpu.VMEM((NUM_CHOICES, HIDDEN_DIM), jnp.float32),
            pltpu.SemaphoreType.DMA,
        ],
        compiler_params=pltpu.CompilerParams(
            dimension_semantics=("arbitrary",),
        ),
    )(x, W, b2)
    return out_t.T


# speedup vs baseline: 1.4525x; 1.0483x over previous
import jax
import jax.numpy as jnp
from jax.experimental import pallas as pl
from jax.experimental.pallas import tpu as pltpu

HIDDEN_DIM = 2048
NUM_CHOICES = 64
BLOCK_M = 1024


N_CHUNKS = 4
CHUNK = BLOCK_M // N_CHUNKS


def _gate_kernel(x_ref, w_ref, b_ref, o_ref):
    w = w_ref[...]
    b = b_ref[...]
    for h in range(N_CHUNKS):
        logits_t = jax.lax.dot_general(
            w, x_ref[pl.ds(h * CHUNK, CHUNK), :],
            dimension_numbers=(((1,), (1,)), ((), ())),
            preferred_element_type=jnp.float32,
        )
        logits_t = logits_t + b
        col_max = jnp.max(logits_t, axis=0, keepdims=True)
        row = jax.lax.broadcasted_iota(jnp.int32, logits_t.shape, 0)
        cand = jnp.where(logits_t == col_max, row, NUM_CHOICES)
        idx = jnp.min(cand, axis=0, keepdims=True)
        o_ref[:, pl.ds(h * CHUNK, CHUNK)] = (row == idx).astype(o_ref.dtype)


def kernel(x, W, b):
    n = x.shape[0]
    b2 = b.reshape(NUM_CHOICES, 1)
    out_t = pl.pallas_call(
        _gate_kernel,
        grid=(n // BLOCK_M,),
        in_specs=[
            pl.BlockSpec((BLOCK_M, HIDDEN_DIM), lambda i: (i, 0)),
            pl.BlockSpec((NUM_CHOICES, HIDDEN_DIM), lambda i: (0, 0)),
            pl.BlockSpec((NUM_CHOICES, 1), lambda i: (0, 0)),
        ],
        out_specs=pl.BlockSpec((NUM_CHOICES, BLOCK_M), lambda i: (0, i)),
        out_shape=jax.ShapeDtypeStruct((NUM_CHOICES, n), x.dtype),
        compiler_params=pltpu.CompilerParams(
            dimension_semantics=("arbitrary",),
        ),
    )(x, W, b2)
    return out_t.T
